# Initial kernel scaffold; baseline (speedup 1.0000x reference)
import jax
import jax.numpy as jnp
from jax.experimental import pallas as pl


def kernel(x, edge_index, edge_attr, params):
    raise NotImplementedError("write your pallas kernel here")


# speedup vs baseline: 3.5864x; 3.5864x over previous
import functools

import jax
import jax.numpy as jnp
from jax import lax
from jax.experimental import pallas as pl
from jax.experimental.pallas import tpu as pltpu
from jax.experimental.pallas import tpu_sc as plsc

_N = 10000
_E = 320000
_D = 128

_NC, _NS = 2, 16
_NW = _NC * _NS
_ROW = 128
_NROWS = _E // _ROW
_RPW = -(-_NROWS // _NW)
_ZR = 200
_NZCH = _N // _ZR
_ZPW = -(-_NZCH // _NS)

_EBLK = 2560
_NBLK = 2000


def _ln_rows(d, scale, offset):
    m = jnp.mean(d, axis=-1, keepdims=True)
    c = d - m
    v = jnp.mean(c * c, axis=-1, keepdims=True)
    return c * lax.rsqrt(v + 1e-5) * scale + offset


def _row(a):
    return a.reshape(1, -1)


def _mlp_ln(x, p, blk):
    n, din = x.shape

    def body(x_ref, w0, b0, w1, b1, sc, of, o_ref):
        h = jnp.maximum(
            jnp.dot(x_ref[...], w0[...], preferred_element_type=jnp.float32)
            + b0[...], 0.0)
        d = (jnp.dot(h, w1[...], preferred_element_type=jnp.float32)
             + b1[...])
        o_ref[...] = _ln_rows(d, sc[...], of[...])

    return pl.pallas_call(
        body,
        grid=(n // blk,),
        in_specs=[
            pl.BlockSpec((blk, din), lambda i: (i, 0)),
            pl.BlockSpec((din, _D), lambda i: (0, 0)),
            pl.BlockSpec((1, _D), lambda i: (0, 0)),
            pl.BlockSpec((_D, _D), lambda i: (0, 0)),
            pl.BlockSpec((1, _D), lambda i: (0, 0)),
            pl.BlockSpec((1, _D), lambda i: (0, 0)),
            pl.BlockSpec((1, _D), lambda i: (0, 0)),
        ],
        out_specs=pl.BlockSpec((blk, _D), lambda i: (i, 0)),
        out_shape=jax.ShapeDtypeStruct((n, _D), jnp.float32),
    )(x, p["w0"], _row(p["b0"]), p["w1"], _row(p["b1"]),
      _row(p["scale"]), _row(p["offset"]))


def _decode(v, p):
    def body(x_ref, w0, b0, w1, b1, o_ref):
        h = jnp.maximum(
            jnp.dot(x_ref[...], w0[...], preferred_element_type=jnp.float32)
            + b0[...], 0.0)
        o_ref[...] = (jnp.dot(h, w1[...], preferred_element_type=jnp.float32)
                      + b1[...])

    return pl.pallas_call(
        body,
        grid=(_N // _NBLK,),
        in_specs=[
            pl.BlockSpec((_NBLK, _D), lambda i: (i, 0)),
            pl.BlockSpec((_D, _D), lambda i: (0, 0)),
            pl.BlockSpec((1, _D), lambda i: (0, 0)),
            pl.BlockSpec((_D, _D), lambda i: (0, 0)),
            pl.BlockSpec((1, _D), lambda i: (0, 0)),
        ],
        out_specs=pl.BlockSpec((_NBLK, _D), lambda i: (i, 0)),
        out_shape=jax.ShapeDtypeStruct((_N, _D), jnp.float32),
    )(v, p["w0"], _row(p["b0"]), p["w1"], _row(p["b1"]))


def _project(v, w0s, w0r):
    wst = jnp.stack([w0s, w0r])

    def body(v_ref, w_ref, o_ref):
        o_ref[...] = jnp.dot(
            v_ref[...], w_ref[0], preferred_element_type=jnp.float32)[None]

    out = pl.pallas_call(
        body,
        grid=(2, _N // _NBLK),
        in_specs=[
            pl.BlockSpec((_NBLK, _D), lambda g, i: (i, 0)),
            pl.BlockSpec((1, _D, _D), lambda g, i: (g, 0, 0)),
        ],
        out_specs=pl.BlockSpec((1, _NBLK, _D), lambda g, i: (g, i, 0)),
        out_shape=jax.ShapeDtypeStruct((2, _N, _D), jnp.float32),
    )(v, wst)
    return out[0], out[1]


def _edge_update(e, gs, gr, p):
    w0e = p["w0"][:_D]

    def body(e_ref, gs_ref, gr_ref, w0, b0, w1, b1, sc, of, o_ref):
        ee = e_ref[...]
        pre = (jnp.dot(ee, w0[...], preferred_element_type=jnp.float32)
               + gs_ref[...] + gr_ref[...] + b0[...])
        h = jnp.maximum(pre, 0.0)
        d = (jnp.dot(h, w1[...], preferred_element_type=jnp.float32)
             + b1[...])
        o_ref[...] = ee + _ln_rows(d, sc[...], of[...])

    return pl.pallas_call(
        body,
        grid=(_E // _EBLK,),
        in_specs=[
            pl.BlockSpec((_EBLK, _D), lambda i: (i, 0)),
            pl.BlockSpec((_EBLK, _D), lambda i: (i, 0)),
            pl.BlockSpec((_EBLK, _D), lambda i: (i, 0)),
            pl.BlockSpec((_D, _D), lambda i: (0, 0)),
            pl.BlockSpec((1, _D), lambda i: (0, 0)),
            pl.BlockSpec((_D, _D), lambda i: (0, 0)),
            pl.BlockSpec((1, _D), lambda i: (0, 0)),
            pl.BlockSpec((1, _D), lambda i: (0, 0)),
            pl.BlockSpec((1, _D), lambda i: (0, 0)),
        ],
        out_specs=pl.BlockSpec((_EBLK, _D), lambda i: (i, 0)),
        out_shape=jax.ShapeDtypeStruct((_E, _D), jnp.float32),
    )(e, gs, gr, w0e, _row(p["b0"]), p["w1"], _row(p["b1"]),
      _row(p["scale"]), _row(p["offset"]))


def _node_update(v, agg2, p):
    w0a = p["w0"][:_D]
    w0b = p["w0"][_D:]

    def body(v_ref, a_ref, wa, wb, b0, w1, b1, sc, of, o_ref):
        vv = v_ref[...]
        agg = a_ref[0] + a_ref[1]
        pre = (jnp.dot(vv, wa[...], preferred_element_type=jnp.float32)
               + jnp.dot(agg, wb[...], preferred_element_type=jnp.float32)
               + b0[...])
        h = jnp.maximum(pre, 0.0)
        d = (jnp.dot(h, w1[...], preferred_element_type=jnp.float32)
             + b1[...])
        o_ref[...] = vv + _ln_rows(d, sc[...], of[...])

    return pl.pallas_call(
        body,
        grid=(_N // _NBLK,),
        in_specs=[
            pl.BlockSpec((_NBLK, _D), lambda i: (i, 0)),
            pl.BlockSpec((2, _NBLK, _D), lambda i: (0, i, 0)),
            pl.BlockSpec((_D, _D), lambda i: (0, 0)),
            pl.BlockSpec((_D, _D), lambda i: (0, 0)),
            pl.BlockSpec((1, _D), lambda i: (0, 0)),
            pl.BlockSpec((_D, _D), lambda i: (0, 0)),
            pl.BlockSpec((1, _D), lambda i: (0, 0)),
            pl.BlockSpec((1, _D), lambda i: (0, 0)),
            pl.BlockSpec((1, _D), lambda i: (0, 0)),
        ],
        out_specs=pl.BlockSpec((_NBLK, _D), lambda i: (i, 0)),
        out_shape=jax.ShapeDtypeStruct((_N, _D), jnp.float32),
    )(v, agg2, w0a, w0b, _row(p["b0"]), p["w1"], _row(p["b1"]),
      _row(p["scale"]), _row(p["offset"]))


def _sc_mesh():
    return plsc.VectorSubcoreMesh(core_axis_name="c", subcore_axis_name="s")


def _sc_gather(ps, pr, sidx, ridx):
    @functools.partial(
        pl.kernel,
        mesh=_sc_mesh(),
        out_type=(jax.ShapeDtypeStruct((_E, _D), jnp.float32),
                  jax.ShapeDtypeStruct((_E, _D), jnp.float32)),
        scratch_types=[
            pltpu.VMEM((_ROW,), jnp.int32),
            pltpu.VMEM((_ROW,), jnp.int32),
            pltpu.VMEM((_ROW, _D), jnp.float32),
            pltpu.VMEM((_ROW, _D), jnp.float32),
            pltpu.SemaphoreType.DMA,
            pltpu.SemaphoreType.DMA,
        ],
    )
    def k(ps_hbm, pr_hbm, s_hbm, r_hbm, gs_hbm, gr_hbm,
          sidx_v, ridx_v, gs_v, gr_v, sem1, sem2):
        wid = lax.axis_index("s") * _NC + lax.axis_index("c")

        def body(i, carry):
            r = i * _NW + wid

            @pl.when(r < _NROWS)
            def _():
                base = r * _ROW
                pltpu.sync_copy(s_hbm.at[pl.ds(base, _ROW)], sidx_v)
                pltpu.sync_copy(r_hbm.at[pl.ds(base, _ROW)], ridx_v)
                c1 = pltpu.async_copy(ps_hbm.at[sidx_v], gs_v, sem1)
                c2 = pltpu.async_copy(pr_hbm.at[ridx_v], gr_v, sem2)
                c1.wait()
                c2.wait()
                pltpu.sync_copy(gs_v, gs_hbm.at[pl.ds(base, _ROW)])
                pltpu.sync_copy(gr_v, gr_hbm.at[pl.ds(base, _ROW)])

            return carry

        lax.fori_loop(0, _RPW, body, 0)

    return k(ps, pr, sidx, ridx)


def _sc_scatter(e, ridx):
    @functools.partial(
        pl.kernel,
        mesh=_sc_mesh(),
        out_type=jax.ShapeDtypeStruct((_NC, _N, _D), jnp.float32),
        scratch_types=[
            pltpu.VMEM((_ROW,), jnp.int32),
            pltpu.VMEM((_ROW, _D), jnp.float32),
            pltpu.VMEM((_ZR, _D), jnp.float32),
            pltpu.VMEM_SHARED((_N, _D), jnp.float32),
        ],
    )
    def k(e_hbm, r_hbm, out_hbm, ridx_v, e_v, z_v, agg_sh):
        cid = lax.axis_index("c")
        sid = lax.axis_index("s")
        wid = sid * _NC + cid

        def zb(t, c):
            z_v[t // 8, pl.ds((t % 8) * 16, 16)] = jnp.zeros((16,), jnp.float32)
            return c

        lax.fori_loop(0, _ZR * 8, zb, 0)

        def zs(j, c):
            cc = j * _NS + sid

            @pl.when(cc < _NZCH)
            def _():
                pltpu.sync_copy(z_v, agg_sh.at[pl.ds(cc * _ZR, _ZR)])

            return c

        lax.fori_loop(0, _ZPW, zs, 0)
        plsc.subcore_barrier()

        def body(i, c):
            r = i * _NW + wid

            @pl.when(r < _NROWS)
            def _():
                base = r * _ROW
                pltpu.sync_copy(r_hbm.at[pl.ds(base, _ROW)], ridx_v)
                pltpu.sync_copy(e_hbm.at[pl.ds(base, _ROW)], e_v)
                pltpu.sync_copy(e_v, agg_sh.at[ridx_v], add=True)

            return c

        lax.fori_loop(0, _RPW, body, 0)
        plsc.subcore_barrier()

        def dump(j, c):
            cc = j * _NS + sid

            @pl.when(cc < _NZCH)
            def _():
                sl = pl.ds(cc * _ZR, _ZR)
                pltpu.sync_copy(agg_sh.at[sl], z_v)
                pltpu.sync_copy(z_v, out_hbm.at[cid, sl])

            return c

        lax.fori_loop(0, _ZPW, dump, 0)

    return k(e, ridx)


def kernel(x, edge_index, edge_attr, params):
    senders = edge_index[0]
    receivers = edge_index[1]

    v = _mlp_ln(x, params["enc_node"], _NBLK)
    e = _mlp_ln(edge_attr, params["enc_edge"], _EBLK)

    for step in params["proc"]:
        pe = step["edge"]
        pn = step["node"]
        ps, pr = _project(v, pe["w0"][_D:2 * _D], pe["w0"][2 * _D:])
        gs, gr = _sc_gather(ps, pr, senders, receivers)
        e = _edge_update(e, gs, gr, pe)
        agg2 = _sc_scatter(e, receivers)
        v = _node_update(v, agg2, pn)

    return _decode(v, params["dec_node"])
